# baseline (device time: 157859 ns/iter reference)
import jax
import jax.numpy as jnp
from jax import lax
from jax.experimental import pallas as pl
from jax.experimental.pallas import tpu as pltpu

N_DEV = 4
E_LOCAL = 4
E_TOT = 16
CAP = 128
BLK = E_LOCAL * CAP


def _pallas_core(xs, x, expert_W, shared_W):
    n_tok, d_model = x.shape
    d_hidden = expert_W.shape[-1]

    def body(xs_ref, x_ref, ew_ref, sw_ref, sh_out_ref, y_out_ref,
             recv_xs, y_stage, sem_sx, sem_rx, sem_sy, sem_ry):
        me = lax.axis_index("i")

        bar = pltpu.get_barrier_semaphore()
        for d in range(1, N_DEV):
            pl.semaphore_signal(
                bar, inc=1,
                device_id=((me + d) % N_DEV,),
                device_id_type=pl.DeviceIdType.MESH,
            )
        pl.semaphore_wait(bar, N_DEV - 1)

        sends = []
        for d in range(1, N_DEV):
            q = (me + d) % N_DEV
            r = pltpu.make_async_remote_copy(
                src_ref=xs_ref.at[pl.ds(q * BLK, BLK), :],
                dst_ref=recv_xs.at[me],
                send_sem=sem_sx.at[q], recv_sem=sem_rx.at[me],
                device_id=(q,), device_id_type=pl.DeviceIdType.MESH,
            )
            r.start()
            sends.append(r)

        w_bf = ew_ref[...].astype(jnp.bfloat16)

        def experts(xb):
            outs = []
            for k in range(E_LOCAL):
                outs.append(jnp.dot(
                    xb[k * CAP:(k + 1) * CAP, :], w_bf[k],
                    preferred_element_type=jnp.float32))
            return jnp.concatenate(outs, axis=0)

        x_bf = x_ref[...].astype(jnp.bfloat16)
        sw_bf = sw_ref[...].astype(jnp.bfloat16)
        sh_out_ref[...] = jnp.dot(x_bf, sw_bf,
                                  preferred_element_type=jnp.float32)
        own = experts(xs_ref[pl.ds(me * BLK, BLK), :])
        y_out_ref[pl.ds(me * BLK, BLK), :] = own.astype(jnp.bfloat16)

        for d in (1, 3, 2):
            q = (me + d) % N_DEV
            w = pltpu.make_async_remote_copy(
                src_ref=recv_xs.at[q], dst_ref=recv_xs.at[q],
                send_sem=sem_rx.at[q], recv_sem=sem_rx.at[q],
                device_id=(q,), device_id_type=pl.DeviceIdType.MESH,
            )
            w.wait_recv()
            y_stage[q, :, :] = experts(recv_xs[q]).astype(jnp.bfloat16)
            r = pltpu.make_async_remote_copy(
                src_ref=y_stage.at[q],
                dst_ref=y_out_ref.at[pl.ds(me * BLK, BLK), :],
                send_sem=sem_sy.at[q], recv_sem=sem_ry.at[me],
                device_id=(q,), device_id_type=pl.DeviceIdType.MESH,
            )
            r.start()
            sends.append(r)

        for d in (1, 2, 3):
            q = (me + d) % N_DEV
            w = pltpu.make_async_remote_copy(
                src_ref=y_stage.at[q],
                dst_ref=y_out_ref.at[pl.ds(q * BLK, BLK), :],
                send_sem=sem_ry.at[q], recv_sem=sem_ry.at[q],
                device_id=(q,), device_id_type=pl.DeviceIdType.MESH,
            )
            w.wait_recv()
        for r in sends:
            r.wait_send()

    return pl.pallas_call(
        body,
        out_shape=(
            jax.ShapeDtypeStruct((n_tok, d_hidden), jnp.float32),
            jax.ShapeDtypeStruct((N_DEV * BLK, d_hidden), jnp.bfloat16),
        ),
        in_specs=[pl.BlockSpec(memory_space=pltpu.VMEM)] * 4,
        out_specs=(pl.BlockSpec(memory_space=pltpu.VMEM),
                   pl.BlockSpec(memory_space=pltpu.VMEM)),
        scratch_shapes=[
            pltpu.VMEM((N_DEV, BLK, d_model), jnp.bfloat16),
            pltpu.VMEM((N_DEV, BLK, d_hidden), jnp.bfloat16),
            pltpu.SemaphoreType.DMA((N_DEV,)),
            pltpu.SemaphoreType.DMA((N_DEV,)),
            pltpu.SemaphoreType.DMA((N_DEV,)),
            pltpu.SemaphoreType.DMA((N_DEV,)),
        ],
        compiler_params=pltpu.CompilerParams(
            collective_id=0, vmem_limit_bytes=100 * 1024 * 1024,
        ),
    )(xs, x, expert_W, shared_W)


def kernel(x, router_W, route_idx, expert_W, shared_W):
    n_tok = x.shape[0]
    route = route_idx[:, 0]

    s = x @ router_W
    s = s - jnp.max(s, axis=1, keepdims=True)
    p = jnp.exp(s)
    p = p / jnp.sum(p, axis=1, keepdims=True)
    gate = jnp.take_along_axis(p, route[:, None], axis=1)

    order = jnp.argsort(route)
    rank = jnp.argsort(order)
    counts = jnp.sum(route[:, None] == jnp.arange(E_TOT)[None, :], axis=0)
    starts = jnp.cumsum(counts) - counts

    slot_e = jnp.arange(E_TOT * CAP) // CAP
    slot_c = jnp.arange(E_TOT * CAP) % CAP
    src_pos = jnp.take(starts, slot_e) + slot_c
    valid = slot_c < jnp.take(counts, slot_e)
    tok = jnp.take(order, jnp.clip(src_pos, 0, n_tok - 1))

    x_gated = x * gate
    xs = jnp.take(x_gated, tok, axis=0) * valid[:, None]
    xs = xs.astype(jnp.bfloat16)

    shared, y_stack = _pallas_core(xs, x, expert_W, shared_W)

    pos = rank - jnp.take(starts, route)
    slot = route * CAP + jnp.clip(pos, 0, CAP - 1)
    return shared + jnp.take(y_stack, slot, axis=0).astype(jnp.float32)


# device time: 61209 ns/iter; 2.5790x vs baseline; 2.5790x over previous
import jax
import jax.numpy as jnp
from jax import lax
from jax.experimental import pallas as pl
from jax.experimental.pallas import tpu as pltpu

N_DEV = 4
E_LOCAL = 4
E_TOT = 16
CAP = 128
BLK = E_LOCAL * CAP
N_SLOT = E_TOT * CAP


def kernel(x, router_W, route_idx, expert_W, shared_W):
    n_tok, d_model = x.shape
    d_hidden = expert_W.shape[-1]

    def body(x_ref, rw_ref, idx_ref, ew_ref, sw_ref, out_ref,
             xs_ref, recv_xs, y_stack, y_stage,
             sem_sx, sem_rx, sem_sy, sem_ry):
        me = lax.axis_index("i")

        bar = pltpu.get_barrier_semaphore()
        for d in range(1, N_DEV):
            pl.semaphore_signal(
                bar, inc=1,
                device_id=((me + d) % N_DEV,),
                device_id_type=pl.DeviceIdType.MESH,
            )
        pl.semaphore_wait(bar, N_DEV - 1)

        x32 = x_ref[...]
        s = jnp.dot(x32, rw_ref[...], preferred_element_type=jnp.float32)
        s = s - jnp.max(s, axis=1, keepdims=True)
        p = jnp.exp(s)
        p = p / jnp.sum(p, axis=1, keepdims=True)
        route = idx_ref[...]
        eids = lax.broadcasted_iota(jnp.int32, (n_tok, E_TOT), 1)
        oh = (eids == route).astype(jnp.bfloat16)
        gate = jnp.sum(p * oh.astype(jnp.float32), axis=1, keepdims=True)

        ir = lax.broadcasted_iota(jnp.int32, (n_tok, n_tok), 0)
        ic = lax.broadcasted_iota(jnp.int32, (n_tok, n_tok), 1)
        ltri = (ic < ir).astype(jnp.bfloat16)
        pos = jnp.sum(
            jnp.dot(ltri, oh, preferred_element_type=jnp.float32)
            * oh.astype(jnp.float32), axis=1, keepdims=True)

        rel = jnp.remainder(route // E_LOCAL - me, N_DEV)
        slot = rel * BLK + jnp.remainder(route, E_LOCAL) * CAP \
            + pos.astype(jnp.int32)
        slot = jnp.where(pos < CAP, slot, 2 * N_SLOT)
        sl = lax.broadcasted_iota(jnp.int32, (n_tok, N_SLOT), 1)
        ptm = (sl == slot).astype(jnp.bfloat16)

        x_bf = x32.astype(jnp.bfloat16)
        ptg = ptm * gate.astype(jnp.bfloat16)

        sends = []
        for d in (1, 3, 2):
            q = (me + d) % N_DEV
            xs_ref[d * BLK:(d + 1) * BLK, :] = lax.dot_general(
                ptg[:, d * BLK:(d + 1) * BLK], x_bf,
                (((0,), (0,)), ((), ())),
                preferred_element_type=jnp.float32).astype(jnp.bfloat16)
            r = pltpu.make_async_remote_copy(
                src_ref=xs_ref.at[pl.ds(d * BLK, BLK), :],
                dst_ref=recv_xs.at[me],
                send_sem=sem_sx.at[q], recv_sem=sem_rx.at[me],
                device_id=(q,), device_id_type=pl.DeviceIdType.MESH,
            )
            r.start()
            sends.append(r)

        w_bf = ew_ref[...].astype(jnp.bfloat16)

        def experts(xb):
            outs = []
            for k in range(E_LOCAL):
                outs.append(jnp.dot(
                    xb[k * CAP:(k + 1) * CAP, :], w_bf[k],
                    preferred_element_type=jnp.float32))
            return jnp.concatenate(outs, axis=0)

        xs_ref[0:BLK, :] = lax.dot_general(
            ptg[:, 0:BLK], x_bf, (((0,), (0,)), ((), ())),
            preferred_element_type=jnp.float32).astype(jnp.bfloat16)
        y_stack[0:BLK, :] = experts(xs_ref[0:BLK, :]).astype(jnp.bfloat16)
        sw_bf = sw_ref[...].astype(jnp.bfloat16)
        acc = jnp.dot(x_bf, sw_bf, preferred_element_type=jnp.float32)
        acc = acc + jnp.dot(ptm[:, 0:BLK], y_stack[0:BLK, :],
                            preferred_element_type=jnp.float32)

        for d in (1, 3, 2):
            q = (me + d) % N_DEV
            w = pltpu.make_async_remote_copy(
                src_ref=recv_xs.at[q], dst_ref=recv_xs.at[q],
                send_sem=sem_rx.at[q], recv_sem=sem_rx.at[q],
                device_id=(q,), device_id_type=pl.DeviceIdType.MESH,
            )
            w.wait_recv()
            y_stage[q, :, :] = experts(recv_xs[q]).astype(jnp.bfloat16)
            r = pltpu.make_async_remote_copy(
                src_ref=y_stage.at[q],
                dst_ref=y_stack.at[pl.ds((N_DEV - d) * BLK, BLK), :],
                send_sem=sem_sy.at[q], recv_sem=sem_ry.at[me],
                device_id=(q,), device_id_type=pl.DeviceIdType.MESH,
            )
            r.start()
            sends.append(r)

        for d in (1, 2, 3):
            q = (me + d) % N_DEV
            w = pltpu.make_async_remote_copy(
                src_ref=y_stage.at[q],
                dst_ref=y_stack.at[pl.ds(d * BLK, BLK), :],
                send_sem=sem_ry.at[q], recv_sem=sem_ry.at[q],
                device_id=(q,), device_id_type=pl.DeviceIdType.MESH,
            )
            w.wait_recv()
            acc = acc + jnp.dot(ptm[:, d * BLK:(d + 1) * BLK],
                                y_stack[d * BLK:(d + 1) * BLK, :],
                                preferred_element_type=jnp.float32)
        out_ref[...] = acc

        for r in sends:
            r.wait_send()

    return pl.pallas_call(
        body,
        out_shape=jax.ShapeDtypeStruct((n_tok, d_hidden), jnp.float32),
        in_specs=[pl.BlockSpec(memory_space=pltpu.VMEM)] * 5,
        out_specs=pl.BlockSpec(memory_space=pltpu.VMEM),
        scratch_shapes=[
            pltpu.VMEM((N_SLOT, d_model), jnp.bfloat16),
            pltpu.VMEM((N_DEV, BLK, d_model), jnp.bfloat16),
            pltpu.VMEM((N_SLOT, d_hidden), jnp.bfloat16),
            pltpu.VMEM((N_DEV, BLK, d_hidden), jnp.bfloat16),
            pltpu.SemaphoreType.DMA((N_DEV,)),
            pltpu.SemaphoreType.DMA((N_DEV,)),
            pltpu.SemaphoreType.DMA((N_DEV,)),
            pltpu.SemaphoreType.DMA((N_DEV,)),
        ],
        compiler_params=pltpu.CompilerParams(
            collective_id=0, vmem_limit_bytes=100 * 1024 * 1024,
        ),
    )(x, router_W, route_idx, expert_W, shared_W)


# device time: 59963 ns/iter; 2.6326x vs baseline; 1.0208x over previous
import jax
import jax.numpy as jnp
from jax import lax
from jax.experimental import pallas as pl
from jax.experimental.pallas import tpu as pltpu

N_DEV = 4
E_LOCAL = 4
E_TOT = 16
CAP = 128
BLK = E_LOCAL * CAP
N_SLOT = E_TOT * CAP


def kernel(x, router_W, route_idx, expert_W, shared_W):
    n_tok, d_model = x.shape
    d_hidden = expert_W.shape[-1]

    def body(x_ref, rw_ref, idx_ref, ew_ref, sw_ref, out_ref,
             xs_ref, recv_xs, y_stack, y_stage,
             sem_sx, sem_rx, sem_sy, sem_ry):
        me = lax.axis_index("i")

        bar = pltpu.get_barrier_semaphore()
        for d in range(1, N_DEV):
            pl.semaphore_signal(
                bar, inc=1,
                device_id=((me + d) % N_DEV,),
                device_id_type=pl.DeviceIdType.MESH,
            )
        pl.semaphore_wait(bar, N_DEV - 1)

        x32 = x_ref[...]
        s = jnp.dot(x32, rw_ref[...], preferred_element_type=jnp.float32)
        s = s - jnp.max(s, axis=1, keepdims=True)
        p = jnp.exp(s)
        p = p / jnp.sum(p, axis=1, keepdims=True)
        route = idx_ref[...]
        eids = lax.broadcasted_iota(jnp.int32, (n_tok, E_TOT), 1)
        oh = (eids == route).astype(jnp.bfloat16)
        gate = jnp.sum(p * oh.astype(jnp.float32), axis=1, keepdims=True)

        ir = lax.broadcasted_iota(jnp.int32, (n_tok, n_tok), 0)
        ic = lax.broadcasted_iota(jnp.int32, (n_tok, n_tok), 1)
        ltri = (ic < ir).astype(jnp.bfloat16)
        pos = jnp.sum(
            jnp.dot(ltri, oh, preferred_element_type=jnp.float32)
            * oh.astype(jnp.float32), axis=1, keepdims=True)

        rel = jnp.remainder(route // E_LOCAL - me, N_DEV)
        slot = rel * BLK + jnp.remainder(route, E_LOCAL) * CAP \
            + pos.astype(jnp.int32)
        slot = jnp.where(pos < CAP, slot, 2 * N_SLOT)
        sl = lax.broadcasted_iota(jnp.int32, (n_tok, N_SLOT), 1)
        ptm = (sl == slot).astype(jnp.bfloat16)

        x_bf = x32.astype(jnp.bfloat16)
        ptg = ptm * gate.astype(jnp.bfloat16)

        sends = []
        for d in (1, 3, 2):
            q = (me + d) % N_DEV
            xs_ref[d * BLK:(d + 1) * BLK, :] = lax.dot_general(
                ptg[:, d * BLK:(d + 1) * BLK], x_bf,
                (((0,), (0,)), ((), ())),
                preferred_element_type=jnp.float32).astype(jnp.bfloat16)
            r = pltpu.make_async_remote_copy(
                src_ref=xs_ref.at[pl.ds(d * BLK, BLK), :],
                dst_ref=recv_xs.at[me],
                send_sem=sem_sx.at[q], recv_sem=sem_rx.at[me],
                device_id=(q,), device_id_type=pl.DeviceIdType.MESH,
            )
            r.start()
            sends.append(r)

        w_bf = ew_ref[...].astype(jnp.bfloat16)

        def experts(xb):
            outs = []
            for k in range(E_LOCAL):
                outs.append(jnp.dot(
                    xb[k * CAP:(k + 1) * CAP, :], w_bf[k],
                    preferred_element_type=jnp.float32))
            return jnp.concatenate(outs, axis=0)

        xs_ref[0:BLK, :] = lax.dot_general(
            ptg[:, 0:BLK], x_bf, (((0,), (0,)), ((), ())),
            preferred_element_type=jnp.float32).astype(jnp.bfloat16)
        y_stack[0:BLK, :] = experts(xs_ref[0:BLK, :]).astype(jnp.bfloat16)
        sw_bf = sw_ref[...].astype(jnp.bfloat16)
        acc = jnp.dot(x_bf, sw_bf, preferred_element_type=jnp.float32)
        acc = acc + jnp.dot(ptm[:, 0:BLK], y_stack[0:BLK, :],
                            preferred_element_type=jnp.float32)

        for d in (1, 3, 2):
            q = (me + d) % N_DEV
            w = pltpu.make_async_remote_copy(
                src_ref=recv_xs.at[q], dst_ref=recv_xs.at[q],
                send_sem=sem_rx.at[q], recv_sem=sem_rx.at[q],
                device_id=(q,), device_id_type=pl.DeviceIdType.MESH,
            )
            w.wait_recv()
            y_stage[q, :, :] = experts(recv_xs[q]).astype(jnp.bfloat16)
            r = pltpu.make_async_remote_copy(
                src_ref=y_stage.at[q],
                dst_ref=y_stack.at[pl.ds((N_DEV - d) * BLK, BLK), :],
                send_sem=sem_sy.at[q], recv_sem=sem_ry.at[me],
                device_id=(q,), device_id_type=pl.DeviceIdType.MESH,
            )
            r.start()
            sends.append(r)

        for d in (1, 3, 2):
            q = (me + d) % N_DEV
            w = pltpu.make_async_remote_copy(
                src_ref=y_stage.at[q],
                dst_ref=y_stack.at[pl.ds(d * BLK, BLK), :],
                send_sem=sem_ry.at[q], recv_sem=sem_ry.at[q],
                device_id=(q,), device_id_type=pl.DeviceIdType.MESH,
            )
            w.wait_recv()
            acc = acc + jnp.dot(ptm[:, d * BLK:(d + 1) * BLK],
                                y_stack[d * BLK:(d + 1) * BLK, :],
                                preferred_element_type=jnp.float32)
        out_ref[...] = acc

        for r in sends:
            r.wait_send()

    return pl.pallas_call(
        body,
        out_shape=jax.ShapeDtypeStruct((n_tok, d_hidden), jnp.float32),
        in_specs=[pl.BlockSpec(memory_space=pltpu.VMEM)] * 5,
        out_specs=pl.BlockSpec(memory_space=pltpu.VMEM),
        scratch_shapes=[
            pltpu.VMEM((N_SLOT, d_model), jnp.bfloat16),
            pltpu.VMEM((N_DEV, BLK, d_model), jnp.bfloat16),
            pltpu.VMEM((N_SLOT, d_hidden), jnp.bfloat16),
            pltpu.VMEM((N_DEV, BLK, d_hidden), jnp.bfloat16),
            pltpu.SemaphoreType.DMA((N_DEV,)),
            pltpu.SemaphoreType.DMA((N_DEV,)),
            pltpu.SemaphoreType.DMA((N_DEV,)),
            pltpu.SemaphoreType.DMA((N_DEV,)),
        ],
        compiler_params=pltpu.CompilerParams(
            collective_id=0, vmem_limit_bytes=100 * 1024 * 1024,
        ),
    )(x, router_W, route_idx, expert_W, shared_W)
